# Initial kernel scaffold; baseline (speedup 1.0000x reference)
import jax
import jax.numpy as jnp
from jax.experimental import pallas as pl


def kernel(x, edge_index, W_l, b_l, W_r):
    raise NotImplementedError("write your pallas kernel here")


# speedup vs baseline: 166.7701x; 166.7701x over previous
import functools

import jax
import jax.numpy as jnp
from jax import lax
from jax.experimental import pallas as pl
from jax.experimental.pallas import tpu as pltpu
from jax.experimental.pallas import tpu_sc as plsc

_NC = 2
_NS = 16
_NW = _NC * _NS

_LANES = 128
_CHUNK = 12800
_NZCH = 10


@functools.lru_cache(maxsize=None)
def _build_sc_scatter(N, E):
    assert E % _CHUNK == 0
    n_chunks = E // _CHUNK
    k_iters = (n_chunks + _NW - 1) // _NW
    assert N % _NZCH == 0
    zlen = N // _NZCH
    assert zlen % 8 == 0

    mesh = plsc.VectorSubcoreMesh(core_axis_name="c", subcore_axis_name="s")

    @functools.partial(
        pl.kernel,
        mesh=mesh,
        out_type=[
            jax.ShapeDtypeStruct((N,), jnp.float32),
            jax.ShapeDtypeStruct((N,), jnp.float32),
            jax.ShapeDtypeStruct((N,), jnp.float32),
            jax.ShapeDtypeStruct((N,), jnp.float32),
        ],
        scratch_types=[
            pltpu.VMEM((_CHUNK,), jnp.int32),
            pltpu.VMEM((_CHUNK,), jnp.int32),
            pltpu.VMEM((_CHUNK,), jnp.float32),
            pltpu.VMEM((_CHUNK,), jnp.float32),
            pltpu.VMEM_SHARED((N,), jnp.float32),
            pltpu.VMEM_SHARED((N,), jnp.float32),
            pltpu.VMEM_SHARED((N,), jnp.float32),
            pltpu.SemaphoreType.DMA,
        ],
    )
    def sc_scatter(x_hbm, edges_hbm, ones_hbm, zeros_hbm,
                   sum0_out, sum1_out, cnt0_out, cnt1_out,
                   src_v, dst_v, val_v, ones_v, x_sp, acc_s, acc_c, sem):
        c = lax.axis_index("c")
        s = lax.axis_index("s")
        wid = c * _NS + s

        pltpu.sync_copy(ones_hbm, ones_v)

        @pl.when(s < _NZCH)
        def _():
            off = s * zlen
            pltpu.sync_copy(zeros_hbm, val_v.at[pl.ds(0, zlen)])
            pltpu.sync_copy(val_v.at[pl.ds(0, zlen)], acc_s.at[pl.ds(off, zlen)])
            pltpu.sync_copy(val_v.at[pl.ds(0, zlen)], acc_c.at[pl.ds(off, zlen)])
            pltpu.sync_copy(x_hbm.at[pl.ds(off, zlen)], val_v.at[pl.ds(0, zlen)])
            pltpu.sync_copy(val_v.at[pl.ds(0, zlen)], x_sp.at[pl.ds(off, zlen)])

        plsc.subcore_barrier()

        def do_chunk(k, carry):
            g = k * _NW + wid

            @pl.when(g < n_chunks)
            def _():
                pltpu.sync_copy(edges_hbm.at[0, g], src_v)
                pltpu.sync_copy(edges_hbm.at[1, g], dst_v)
                pltpu.async_copy(x_sp.at[src_v], val_v, sem).wait()
                pltpu.sync_copy(val_v, acc_s.at[dst_v], add=True)
                pltpu.sync_copy(ones_v, acc_c.at[dst_v], add=True)
            return carry
        lax.fori_loop(0, k_iters, do_chunk, 0)

        plsc.subcore_barrier()

        @pl.when(s < _NZCH)
        def _():
            off = s * zlen

            pltpu.sync_copy(acc_s.at[pl.ds(off, zlen)], val_v.at[pl.ds(0, zlen)])

            @pl.when(c == 0)
            def _():
                pltpu.sync_copy(val_v.at[pl.ds(0, zlen)], sum0_out.at[pl.ds(off, zlen)])

            @pl.when(c == 1)
            def _():
                pltpu.sync_copy(val_v.at[pl.ds(0, zlen)], sum1_out.at[pl.ds(off, zlen)])

            pltpu.sync_copy(acc_c.at[pl.ds(off, zlen)], val_v.at[pl.ds(0, zlen)])

            @pl.when(c == 0)
            def _():
                pltpu.sync_copy(val_v.at[pl.ds(0, zlen)], cnt0_out.at[pl.ds(off, zlen)])

            @pl.when(c == 1)
            def _():
                pltpu.sync_copy(val_v.at[pl.ds(0, zlen)], cnt1_out.at[pl.ds(off, zlen)])

    return sc_scatter


@functools.lru_cache(maxsize=None)
def _build_tc_dense(N):
    assert (N * 16) % _LANES == 0 and N % 8 == 0
    R = (N * 16) // _LANES

    def body(s0_ref, s1_ref, c0_ref, c1_ref, x_ref, k_ref, wl_ref, bl_ref,
             wr_ref, out_ref):
        ssum = s0_ref[...] + s1_ref[...]
        cnt = c0_ref[...] + c1_ref[...]
        aggr = ssum / jnp.maximum(cnt, 1.0)
        hi = jax.lax.Precision.HIGHEST
        rep = jax.lax.dot(aggr, k_ref[...], precision=hi)
        xrep = jax.lax.dot(x_ref[...], k_ref[...], precision=hi)
        out_ref[...] = jnp.maximum(
            rep * wl_ref[...] + bl_ref[...] + xrep * wr_ref[...], 0.0)

    br = 1600
    grid = (R + br - 1) // br
    vec_spec = pl.BlockSpec((1, _LANES), lambda i: (0, 0))
    col_spec = pl.BlockSpec((br, 8), lambda i: (i, 0))
    return pl.pallas_call(
        body,
        grid=(grid,),
        in_specs=[col_spec, col_spec, col_spec, col_spec, col_spec,
                  pl.BlockSpec((8, _LANES), lambda i: (0, 0)),
                  vec_spec, vec_spec, vec_spec],
        out_specs=pl.BlockSpec((br, _LANES), lambda i: (i, 0)),
        out_shape=jax.ShapeDtypeStruct((R, _LANES), jnp.float32),
    )


def kernel(x, edge_index, W_l, b_l, W_r):
    N, d_in = x.shape
    E = edge_index.shape[1]
    d_out = W_l.shape[0]
    assert d_in == 1 and d_out == 16

    x_flat = x.reshape(N)
    edges_r = edge_index.reshape(2, E // _CHUNK, _CHUNK)
    ones_h = jnp.ones((_CHUNK,), jnp.float32)
    zeros_h = jnp.zeros((N // _NZCH,), jnp.float32)
    s0, s1, c0, c1 = _build_sc_scatter(N, E)(x_flat, edges_r, ones_h, zeros_h)

    R = (N * 16) // _LANES
    s0 = s0.reshape(R, 8)
    s1 = s1.reshape(R, 8)
    c0 = c0.reshape(R, 8)
    c1 = c1.reshape(R, 8)
    x2 = x.reshape(R, 8)
    k_mat = jnp.repeat(jnp.eye(8, dtype=jnp.float32), 16, axis=1)
    wl = jnp.tile(W_l.reshape(-1), 8)[None, :]
    bl = jnp.tile(b_l, 8)[None, :]
    wr = jnp.tile(W_r.reshape(-1), 8)[None, :]
    out_flat = _build_tc_dense(N)(s0, s1, c0, c1, x2, k_mat, wl, bl, wr)
    return out_flat.reshape(N, 16)


# speedup vs baseline: 191.0458x; 1.1456x over previous
import functools

import jax
import jax.numpy as jnp
from jax import lax
from jax.experimental import pallas as pl
from jax.experimental.pallas import tpu as pltpu
from jax.experimental.pallas import tpu_sc as plsc

_NC = 2
_NS = 16
_NW = _NC * _NS

_LANES = 128
_CHUNK = 12800
_NZCH = 10


@functools.lru_cache(maxsize=None)
def _build_sc_scatter(N, E):
    assert E % _CHUNK == 0
    n_chunks = E // _CHUNK
    k_iters = (n_chunks + _NW - 1) // _NW
    assert N % _NZCH == 0
    zlen = N // _NZCH
    assert zlen % 8 == 0

    mesh = plsc.VectorSubcoreMesh(core_axis_name="c", subcore_axis_name="s")

    @functools.partial(
        pl.kernel,
        mesh=mesh,
        out_type=[
            jax.ShapeDtypeStruct((N,), jnp.float32),
            jax.ShapeDtypeStruct((N,), jnp.float32),
            jax.ShapeDtypeStruct((N,), jnp.float32),
            jax.ShapeDtypeStruct((N,), jnp.float32),
        ],
        scratch_types=[
            pltpu.VMEM((_CHUNK,), jnp.int32),
            pltpu.VMEM((_CHUNK,), jnp.int32),
            pltpu.VMEM((_CHUNK,), jnp.float32),
            pltpu.VMEM((_CHUNK,), jnp.float32),
            pltpu.VMEM_SHARED((N,), jnp.float32),
            pltpu.VMEM_SHARED((N,), jnp.float32),
            pltpu.VMEM_SHARED((N,), jnp.float32),
            pltpu.SemaphoreType.DMA,
        ],
    )
    def sc_scatter(x_hbm, srce_hbm, dste_hbm, ones_hbm, zeros_hbm,
                   sum0_out, sum1_out, cnt0_out, cnt1_out,
                   src_v, dst_v, val_v, ones_v, x_sp, acc_s, acc_c, sem):
        c = lax.axis_index("c")
        s = lax.axis_index("s")
        wid = c * _NS + s

        pltpu.sync_copy(ones_hbm, ones_v)

        @pl.when(s < _NZCH)
        def _():
            off = s * zlen
            pltpu.sync_copy(zeros_hbm, val_v.at[pl.ds(0, zlen)])
            pltpu.sync_copy(val_v.at[pl.ds(0, zlen)], acc_s.at[pl.ds(off, zlen)])
            pltpu.sync_copy(val_v.at[pl.ds(0, zlen)], acc_c.at[pl.ds(off, zlen)])
            pltpu.sync_copy(x_hbm.at[pl.ds(off, zlen)], val_v.at[pl.ds(0, zlen)])
            pltpu.sync_copy(val_v.at[pl.ds(0, zlen)], x_sp.at[pl.ds(off, zlen)])

        plsc.subcore_barrier()

        def do_chunk(k, carry):
            g = k * _NW + wid

            @pl.when(g < n_chunks)
            def _():
                pltpu.sync_copy(srce_hbm.at[pl.ds(g * _CHUNK, _CHUNK)], src_v)
                pltpu.sync_copy(dste_hbm.at[pl.ds(g * _CHUNK, _CHUNK)], dst_v)
                pltpu.async_copy(x_sp.at[src_v], val_v, sem).wait()
                pltpu.sync_copy(val_v, acc_s.at[dst_v], add=True)
                pltpu.sync_copy(ones_v, acc_c.at[dst_v], add=True)
            return carry
        lax.fori_loop(0, k_iters, do_chunk, 0)

        plsc.subcore_barrier()

        @pl.when(s < _NZCH)
        def _():
            off = s * zlen

            pltpu.sync_copy(acc_s.at[pl.ds(off, zlen)], val_v.at[pl.ds(0, zlen)])

            @pl.when(c == 0)
            def _():
                pltpu.sync_copy(val_v.at[pl.ds(0, zlen)], sum0_out.at[pl.ds(off, zlen)])

            @pl.when(c == 1)
            def _():
                pltpu.sync_copy(val_v.at[pl.ds(0, zlen)], sum1_out.at[pl.ds(off, zlen)])

            pltpu.sync_copy(acc_c.at[pl.ds(off, zlen)], val_v.at[pl.ds(0, zlen)])

            @pl.when(c == 0)
            def _():
                pltpu.sync_copy(val_v.at[pl.ds(0, zlen)], cnt0_out.at[pl.ds(off, zlen)])

            @pl.when(c == 1)
            def _():
                pltpu.sync_copy(val_v.at[pl.ds(0, zlen)], cnt1_out.at[pl.ds(off, zlen)])

    return sc_scatter


@functools.lru_cache(maxsize=None)
def _build_tc_dense(N):
    assert (N * 16) % _LANES == 0 and N % 8 == 0
    R = (N * 16) // _LANES

    def body(s0_ref, s1_ref, c0_ref, c1_ref, x_ref, k_ref, wl_ref, bl_ref,
             wr_ref, out_ref):
        ssum = s0_ref[...] + s1_ref[...]
        cnt = c0_ref[...] + c1_ref[...]
        aggr = ssum / jnp.maximum(cnt, 1.0)
        hi = jax.lax.Precision.HIGHEST
        rep = jax.lax.dot(aggr, k_ref[...], precision=hi)
        xrep = jax.lax.dot(x_ref[...], k_ref[...], precision=hi)
        out_ref[...] = jnp.maximum(
            rep * wl_ref[...] + bl_ref[...] + xrep * wr_ref[...], 0.0)

    br = 1600
    grid = (R + br - 1) // br
    vec_spec = pl.BlockSpec((1, _LANES), lambda i: (0, 0))
    col_spec = pl.BlockSpec((br, 8), lambda i: (i, 0))
    return pl.pallas_call(
        body,
        grid=(grid,),
        in_specs=[col_spec, col_spec, col_spec, col_spec, col_spec,
                  pl.BlockSpec((8, _LANES), lambda i: (0, 0)),
                  vec_spec, vec_spec, vec_spec],
        out_specs=pl.BlockSpec((br, _LANES), lambda i: (i, 0)),
        out_shape=jax.ShapeDtypeStruct((R, _LANES), jnp.float32),
    )


def kernel(x, edge_index, W_l, b_l, W_r):
    N, d_in = x.shape
    E = edge_index.shape[1]
    d_out = W_l.shape[0]
    assert d_in == 1 and d_out == 16

    x_flat = x.reshape(N)
    src_e = edge_index[0]
    dst_e = edge_index[1]
    ones_h = jnp.ones((_CHUNK,), jnp.float32)
    zeros_h = jnp.zeros((N // _NZCH,), jnp.float32)
    s0, s1, c0, c1 = _build_sc_scatter(N, E)(x_flat, src_e, dst_e, ones_h, zeros_h)

    R = (N * 16) // _LANES
    s0 = s0.reshape(R, 8)
    s1 = s1.reshape(R, 8)
    c0 = c0.reshape(R, 8)
    c1 = c1.reshape(R, 8)
    x2 = x.reshape(R, 8)
    k_mat = jnp.repeat(jnp.eye(8, dtype=jnp.float32), 16, axis=1)
    wl = jnp.tile(W_l.reshape(-1), 8)[None, :]
    bl = jnp.tile(b_l, 8)[None, :]
    wr = jnp.tile(W_r.reshape(-1), 8)[None, :]
    out_flat = _build_tc_dense(N)(s0, s1, c0, c1, x2, k_mat, wl, bl, wr)
    return out_flat.reshape(N, 16)


# speedup vs baseline: 202.9312x; 1.0622x over previous
import functools

import jax
import jax.numpy as jnp
from jax import lax
from jax.experimental import pallas as pl
from jax.experimental.pallas import tpu as pltpu
from jax.experimental.pallas import tpu_sc as plsc

_NC = 2
_NS = 16
_NW = _NC * _NS

_LANES = 128
_CHUNK = 12800
_NZCH = 10


@functools.lru_cache(maxsize=None)
def _build_sc_scatter(N, E):
    assert E % _CHUNK == 0
    n_chunks = E // _CHUNK
    k_iters = (n_chunks + _NW - 1) // _NW
    assert N % _NZCH == 0
    zlen = N // _NZCH
    assert zlen % 8 == 0

    mesh = plsc.VectorSubcoreMesh(core_axis_name="c", subcore_axis_name="s")

    @functools.partial(
        pl.kernel,
        mesh=mesh,
        out_type=[
            jax.ShapeDtypeStruct((N,), jnp.float32),
            jax.ShapeDtypeStruct((N,), jnp.float32),
            jax.ShapeDtypeStruct((N,), jnp.float32),
            jax.ShapeDtypeStruct((N,), jnp.float32),
        ],
        scratch_types=[
            pltpu.VMEM((_CHUNK,), jnp.int32),
            pltpu.VMEM((_CHUNK,), jnp.int32),
            pltpu.VMEM((_CHUNK,), jnp.int32),
            pltpu.VMEM((_CHUNK,), jnp.int32),
            pltpu.VMEM((_CHUNK,), jnp.float32),
            pltpu.VMEM((_CHUNK,), jnp.float32),
            pltpu.VMEM((_CHUNK,), jnp.float32),
            pltpu.VMEM_SHARED((N,), jnp.float32),
            pltpu.VMEM_SHARED((N,), jnp.float32),
            pltpu.VMEM_SHARED((N,), jnp.float32),
            pltpu.SemaphoreType.DMA,
            pltpu.SemaphoreType.DMA,
            pltpu.SemaphoreType.DMA,
            pltpu.SemaphoreType.DMA,
            pltpu.SemaphoreType.DMA,
        ],
    )
    def sc_scatter(x_hbm, srce_hbm, dste_hbm, ones_hbm, zeros_hbm,
                   sum0_out, sum1_out, cnt0_out, cnt1_out,
                   src0, src1, dst0, dst1, val0, val1, ones_v,
                   x_sp, acc_s, acc_c,
                   sem_in0, sem_in1, sem_sc0, sem_sc1, sem_g):
        c = lax.axis_index("c")
        s = lax.axis_index("s")
        wid = c * _NS + s

        slots = ((src0, dst0, val0, sem_in0, sem_sc0),
                 (src1, dst1, val1, sem_in1, sem_sc1))

        pltpu.sync_copy(ones_hbm, ones_v)

        @pl.when(s < _NZCH)
        def _():
            off = s * zlen
            pltpu.sync_copy(zeros_hbm, val0.at[pl.ds(0, zlen)])
            pltpu.sync_copy(val0.at[pl.ds(0, zlen)], acc_s.at[pl.ds(off, zlen)])
            pltpu.sync_copy(val0.at[pl.ds(0, zlen)], acc_c.at[pl.ds(off, zlen)])
            pltpu.sync_copy(x_hbm.at[pl.ds(off, zlen)], val0.at[pl.ds(0, zlen)])
            pltpu.sync_copy(val0.at[pl.ds(0, zlen)], x_sp.at[pl.ds(off, zlen)])

        plsc.subcore_barrier()

        def issue_in(g, slot):
            srcb, dstb, _, semi, _ = slot
            pltpu.async_copy(srce_hbm.at[pl.ds(g * _CHUNK, _CHUNK)], srcb, semi)
            pltpu.async_copy(dste_hbm.at[pl.ds(g * _CHUNK, _CHUNK)], dstb, semi)

        def wait_in(g, slot):
            srcb, dstb, _, semi, _ = slot
            pltpu.make_async_copy(srce_hbm.at[pl.ds(g * _CHUNK, _CHUNK)], srcb, semi).wait()
            pltpu.make_async_copy(dste_hbm.at[pl.ds(g * _CHUNK, _CHUNK)], dstb, semi).wait()

        def wait_sc(slot):
            _, dstb, valb, _, sems = slot
            pltpu.make_async_copy(valb, acc_s.at[dstb], sems).wait()
            pltpu.make_async_copy(ones_v, acc_c.at[dstb], sems).wait()

        def body(k, cur, nxt):
            g = k * _NW + wid
            srcb, dstb, valb, semi, sems = cur

            @pl.when(g < n_chunks)
            def _():
                wait_in(g, cur)
                pltpu.async_copy(ones_v, acc_c.at[dstb], sems, add=True)
                pltpu.async_copy(x_sp.at[srcb], valb, sem_g).wait()
                pltpu.async_copy(valb, acc_s.at[dstb], sems, add=True)

            @pl.when(jnp.logical_and(k >= 1, (g - _NW) < n_chunks))
            def _():
                wait_sc(nxt)

            @pl.when((g + _NW) < n_chunks)
            def _():
                issue_in(g + _NW, nxt)

        issue_in(wid, slots[0])

        def pair(p, carry):
            body(2 * p, slots[0], slots[1])
            body(2 * p + 1, slots[1], slots[0])
            return carry
        assert k_iters % 2 == 0
        lax.fori_loop(0, k_iters // 2, pair, 0)

        last = k_iters - 1

        @pl.when((last * _NW + wid) < n_chunks)
        def _():
            wait_sc(slots[last % 2])

        plsc.subcore_barrier()

        @pl.when(s < _NZCH)
        def _():
            off = s * zlen

            pltpu.sync_copy(acc_s.at[pl.ds(off, zlen)], val0.at[pl.ds(0, zlen)])

            @pl.when(c == 0)
            def _():
                pltpu.sync_copy(val0.at[pl.ds(0, zlen)], sum0_out.at[pl.ds(off, zlen)])

            @pl.when(c == 1)
            def _():
                pltpu.sync_copy(val0.at[pl.ds(0, zlen)], sum1_out.at[pl.ds(off, zlen)])

            pltpu.sync_copy(acc_c.at[pl.ds(off, zlen)], val0.at[pl.ds(0, zlen)])

            @pl.when(c == 0)
            def _():
                pltpu.sync_copy(val0.at[pl.ds(0, zlen)], cnt0_out.at[pl.ds(off, zlen)])

            @pl.when(c == 1)
            def _():
                pltpu.sync_copy(val0.at[pl.ds(0, zlen)], cnt1_out.at[pl.ds(off, zlen)])

    return sc_scatter


@functools.lru_cache(maxsize=None)
def _build_tc_dense(N):
    assert (N * 16) % _LANES == 0 and N % 8 == 0
    R = (N * 16) // _LANES

    def body(s0_ref, s1_ref, c0_ref, c1_ref, x_ref, k_ref, wl_ref, bl_ref,
             wr_ref, out_ref):
        ssum = s0_ref[...] + s1_ref[...]
        cnt = c0_ref[...] + c1_ref[...]
        aggr = ssum / jnp.maximum(cnt, 1.0)
        hi = jax.lax.Precision.HIGHEST
        rep = jax.lax.dot(aggr, k_ref[...], precision=hi)
        xrep = jax.lax.dot(x_ref[...], k_ref[...], precision=hi)
        out_ref[...] = jnp.maximum(
            rep * wl_ref[...] + bl_ref[...] + xrep * wr_ref[...], 0.0)

    br = 1600
    grid = (R + br - 1) // br
    vec_spec = pl.BlockSpec((1, _LANES), lambda i: (0, 0))
    col_spec = pl.BlockSpec((br, 8), lambda i: (i, 0))
    return pl.pallas_call(
        body,
        grid=(grid,),
        in_specs=[col_spec, col_spec, col_spec, col_spec, col_spec,
                  pl.BlockSpec((8, _LANES), lambda i: (0, 0)),
                  vec_spec, vec_spec, vec_spec],
        out_specs=pl.BlockSpec((br, _LANES), lambda i: (i, 0)),
        out_shape=jax.ShapeDtypeStruct((R, _LANES), jnp.float32),
    )


def kernel(x, edge_index, W_l, b_l, W_r):
    N, d_in = x.shape
    E = edge_index.shape[1]
    d_out = W_l.shape[0]
    assert d_in == 1 and d_out == 16

    x_flat = x.reshape(N)
    src_e = edge_index[0]
    dst_e = edge_index[1]
    ones_h = jnp.ones((_CHUNK,), jnp.float32)
    zeros_h = jnp.zeros((N // _NZCH,), jnp.float32)
    s0, s1, c0, c1 = _build_sc_scatter(N, E)(x_flat, src_e, dst_e, ones_h, zeros_h)

    R = (N * 16) // _LANES
    s0 = s0.reshape(R, 8)
    s1 = s1.reshape(R, 8)
    c0 = c0.reshape(R, 8)
    c1 = c1.reshape(R, 8)
    x2 = x.reshape(R, 8)
    k_mat = jnp.repeat(jnp.eye(8, dtype=jnp.float32), 16, axis=1)
    wl = jnp.tile(W_l.reshape(-1), 8)[None, :]
    bl = jnp.tile(b_l, 8)[None, :]
    wr = jnp.tile(W_r.reshape(-1), 8)[None, :]
    out_flat = _build_tc_dense(N)(s0, s1, c0, c1, x2, k_mat, wl, bl, wr)
    return out_flat.reshape(N, 16)


# speedup vs baseline: 229.3634x; 1.1303x over previous
import functools

import jax
import jax.numpy as jnp
from jax import lax
from jax.experimental import pallas as pl
from jax.experimental.pallas import tpu as pltpu
from jax.experimental.pallas import tpu_sc as plsc

_NC = 2
_NS = 16
_NW = _NC * _NS

_LANES = 128
_CHUNK = 6400
_NZCH = 10


@functools.lru_cache(maxsize=None)
def _build_sc_scatter(N, E):
    assert E % _CHUNK == 0
    n_chunks = E // _CHUNK
    k_iters = (n_chunks + _NW - 1) // _NW
    assert N % _NZCH == 0
    zlen = N // _NZCH
    assert zlen % 8 == 0

    mesh = plsc.VectorSubcoreMesh(core_axis_name="c", subcore_axis_name="s")

    @functools.partial(
        pl.kernel,
        mesh=mesh,
        out_type=[
            jax.ShapeDtypeStruct((N,), jnp.float32),
            jax.ShapeDtypeStruct((N,), jnp.float32),
            jax.ShapeDtypeStruct((N,), jnp.float32),
            jax.ShapeDtypeStruct((N,), jnp.float32),
        ],
        scratch_types=[
            pltpu.VMEM((2, _CHUNK), jnp.int32),
            pltpu.VMEM((2, _CHUNK), jnp.int32),
            pltpu.VMEM((_CHUNK,), jnp.int32),
            pltpu.VMEM((_CHUNK,), jnp.int32),
            pltpu.VMEM((_CHUNK,), jnp.int32),
            pltpu.VMEM((_CHUNK,), jnp.int32),
            pltpu.VMEM((_CHUNK,), jnp.float32),
            pltpu.VMEM((_CHUNK,), jnp.float32),
            pltpu.VMEM((_CHUNK,), jnp.float32),
            pltpu.VMEM_SHARED((N,), jnp.float32),
            pltpu.VMEM_SHARED((N,), jnp.float32),
            pltpu.VMEM_SHARED((N,), jnp.float32),
            pltpu.SemaphoreType.DMA,
            pltpu.SemaphoreType.DMA,
            pltpu.SemaphoreType.DMA,
            pltpu.SemaphoreType.DMA,
            pltpu.SemaphoreType.DMA,
        ],
    )
    def sc_scatter(x_hbm, edge_hbm, ones_hbm, zeros_hbm,
                   sum0_out, sum1_out, cnt0_out, cnt1_out,
                   edge0, edge1, src0, src1, dst0, dst1, val0, val1, ones_v,
                   x_sp, acc_s, acc_c,
                   sem_in0, sem_in1, sem_sc0, sem_sc1, sem_g):
        c = lax.axis_index("c")
        s = lax.axis_index("s")
        wid = c * _NS + s

        slots = ((edge0, src0, dst0, val0, sem_in0, sem_sc0),
                 (edge1, src1, dst1, val1, sem_in1, sem_sc1))

        pltpu.sync_copy(ones_hbm, ones_v)

        @pl.when(s < _NZCH)
        def _():
            off = s * zlen
            pltpu.sync_copy(zeros_hbm, val0.at[pl.ds(0, zlen)])
            pltpu.sync_copy(val0.at[pl.ds(0, zlen)], acc_s.at[pl.ds(off, zlen)])
            pltpu.sync_copy(val0.at[pl.ds(0, zlen)], acc_c.at[pl.ds(off, zlen)])
            pltpu.sync_copy(x_hbm.at[pl.ds(off, zlen)], val0.at[pl.ds(0, zlen)])
            pltpu.sync_copy(val0.at[pl.ds(0, zlen)], x_sp.at[pl.ds(off, zlen)])

        plsc.subcore_barrier()

        def issue_in(g, slot):
            edgeb, srcb, _, _, semi, _ = slot
            pltpu.async_copy(edge_hbm.at[0, pl.ds(g * _CHUNK, _CHUNK)], srcb, semi)
            pltpu.async_copy(edge_hbm.at[:, pl.ds(g * _CHUNK, _CHUNK)], edgeb, semi)

        def wait_in(g, slot):
            edgeb, srcb, _, _, semi, _ = slot
            pltpu.make_async_copy(edge_hbm.at[0, pl.ds(g * _CHUNK, _CHUNK)], srcb, semi).wait()
            pltpu.make_async_copy(edge_hbm.at[:, pl.ds(g * _CHUNK, _CHUNK)], edgeb, semi).wait()

        def wait_sc(slot):
            _, _, dstb, valb, _, sems = slot
            pltpu.make_async_copy(valb, acc_s.at[dstb], sems).wait()
            pltpu.make_async_copy(ones_v, acc_c.at[dstb], sems).wait()

        def body(k, cur, nxt):
            g = k * _NW + wid
            edgeb, srcb, dstb, valb, semi, sems = cur

            @pl.when(g < n_chunks)
            def _():
                wait_in(g, cur)

                @plsc.parallel_loop(0, _CHUNK, 16, unroll=8)
                def _(j):
                    dstb[pl.ds(j, 16)] = edgeb[1, pl.ds(j, 16)]

                pltpu.async_copy(ones_v, acc_c.at[dstb], sems, add=True)
                pltpu.async_copy(x_sp.at[srcb], valb, sem_g).wait()
                pltpu.async_copy(valb, acc_s.at[dstb], sems, add=True)

            @pl.when(jnp.logical_and(k >= 1, (g - _NW) < n_chunks))
            def _():
                wait_sc(nxt)

            @pl.when((g + _NW) < n_chunks)
            def _():
                issue_in(g + _NW, nxt)

        issue_in(wid, slots[0])

        def pair(p, carry):
            body(2 * p, slots[0], slots[1])
            body(2 * p + 1, slots[1], slots[0])
            return carry
        assert k_iters % 2 == 0
        lax.fori_loop(0, k_iters // 2, pair, 0)

        last = k_iters - 1

        @pl.when((last * _NW + wid) < n_chunks)
        def _():
            wait_sc(slots[last % 2])

        plsc.subcore_barrier()

        @pl.when(s < _NZCH)
        def _():
            off = s * zlen

            pltpu.sync_copy(acc_s.at[pl.ds(off, zlen)], val0.at[pl.ds(0, zlen)])

            @pl.when(c == 0)
            def _():
                pltpu.sync_copy(val0.at[pl.ds(0, zlen)], sum0_out.at[pl.ds(off, zlen)])

            @pl.when(c == 1)
            def _():
                pltpu.sync_copy(val0.at[pl.ds(0, zlen)], sum1_out.at[pl.ds(off, zlen)])

            pltpu.sync_copy(acc_c.at[pl.ds(off, zlen)], val0.at[pl.ds(0, zlen)])

            @pl.when(c == 0)
            def _():
                pltpu.sync_copy(val0.at[pl.ds(0, zlen)], cnt0_out.at[pl.ds(off, zlen)])

            @pl.when(c == 1)
            def _():
                pltpu.sync_copy(val0.at[pl.ds(0, zlen)], cnt1_out.at[pl.ds(off, zlen)])

    return sc_scatter


@functools.lru_cache(maxsize=None)
def _build_tc_dense(N):
    assert (N * 16) % _LANES == 0 and N % 8 == 0
    R = (N * 16) // _LANES

    def body(s0_ref, s1_ref, c0_ref, c1_ref, x_ref, k_ref, wl_ref, bl_ref,
             wr_ref, out_ref):
        ssum = s0_ref[...] + s1_ref[...]
        cnt = c0_ref[...] + c1_ref[...]
        aggr = ssum / jnp.maximum(cnt, 1.0)
        hi = jax.lax.Precision.HIGHEST
        rep = jax.lax.dot(aggr, k_ref[...], precision=hi)
        xrep = jax.lax.dot(x_ref[...], k_ref[...], precision=hi)
        out_ref[...] = jnp.maximum(
            rep * wl_ref[...] + bl_ref[...] + xrep * wr_ref[...], 0.0)

    br = 1600
    grid = (R + br - 1) // br
    vec_spec = pl.BlockSpec((1, _LANES), lambda i: (0, 0))
    col_spec = pl.BlockSpec((br, 8), lambda i: (i, 0))
    return pl.pallas_call(
        body,
        grid=(grid,),
        in_specs=[col_spec, col_spec, col_spec, col_spec, col_spec,
                  pl.BlockSpec((8, _LANES), lambda i: (0, 0)),
                  vec_spec, vec_spec, vec_spec],
        out_specs=pl.BlockSpec((br, _LANES), lambda i: (i, 0)),
        out_shape=jax.ShapeDtypeStruct((R, _LANES), jnp.float32),
    )


def kernel(x, edge_index, W_l, b_l, W_r):
    N, d_in = x.shape
    E = edge_index.shape[1]
    d_out = W_l.shape[0]
    assert d_in == 1 and d_out == 16

    x_flat = x.reshape(N)
    ones_h = jnp.ones((_CHUNK,), jnp.float32)
    zeros_h = jnp.zeros((N // _NZCH,), jnp.float32)
    s0, s1, c0, c1 = _build_sc_scatter(N, E)(x_flat, edge_index, ones_h, zeros_h)

    R = (N * 16) // _LANES
    s0 = s0.reshape(R, 8)
    s1 = s1.reshape(R, 8)
    c0 = c0.reshape(R, 8)
    c1 = c1.reshape(R, 8)
    x2 = x.reshape(R, 8)
    k_mat = jnp.repeat(jnp.eye(8, dtype=jnp.float32), 16, axis=1)
    wl = jnp.tile(W_l.reshape(-1), 8)[None, :]
    bl = jnp.tile(b_l, 8)[None, :]
    wr = jnp.tile(W_r.reshape(-1), 8)[None, :]
    out_flat = _build_tc_dense(N)(s0, s1, c0, c1, x2, k_mat, wl, bl, wr)
    return out_flat.reshape(N, 16)


# speedup vs baseline: 230.6336x; 1.0055x over previous
import functools

import jax
import jax.numpy as jnp
from jax import lax
from jax.experimental import pallas as pl
from jax.experimental.pallas import tpu as pltpu
from jax.experimental.pallas import tpu_sc as plsc

_NC = 2
_NS = 16
_NW = _NC * _NS

_LANES = 128
_CHUNK = 6400
_NZCH = 10


@functools.lru_cache(maxsize=None)
def _build_sc_scatter(N, E):
    assert E % _CHUNK == 0
    n_chunks = E // _CHUNK
    k_iters = (n_chunks + _NW - 1) // _NW
    assert N % _NZCH == 0
    zlen = N // _NZCH
    assert zlen % 8 == 0

    mesh = plsc.VectorSubcoreMesh(core_axis_name="c", subcore_axis_name="s")

    @functools.partial(
        pl.kernel,
        mesh=mesh,
        out_type=[
            jax.ShapeDtypeStruct((N,), jnp.float32),
            jax.ShapeDtypeStruct((N,), jnp.float32),
            jax.ShapeDtypeStruct((N,), jnp.float32),
            jax.ShapeDtypeStruct((N,), jnp.float32),
        ],
        scratch_types=[
            pltpu.VMEM((2, _CHUNK), jnp.int32),
            pltpu.VMEM((2, _CHUNK), jnp.int32),
            pltpu.VMEM((_CHUNK,), jnp.int32),
            pltpu.VMEM((_CHUNK,), jnp.int32),
            pltpu.VMEM((_CHUNK,), jnp.int32),
            pltpu.VMEM((_CHUNK,), jnp.int32),
            pltpu.VMEM((_CHUNK,), jnp.float32),
            pltpu.VMEM((_CHUNK,), jnp.float32),
            pltpu.VMEM((_CHUNK,), jnp.float32),
            pltpu.VMEM_SHARED((N,), jnp.float32),
            pltpu.VMEM_SHARED((N,), jnp.float32),
            pltpu.VMEM_SHARED((N,), jnp.float32),
            pltpu.SemaphoreType.DMA,
            pltpu.SemaphoreType.DMA,
            pltpu.SemaphoreType.DMA,
            pltpu.SemaphoreType.DMA,
            pltpu.SemaphoreType.DMA,
        ],
        compiler_params=pltpu.CompilerParams(use_tc_tiling_on_sc=True),
    )
    def sc_scatter(x_hbm, edge_hbm, ones_hbm, zeros_hbm,
                   sum0_out, sum1_out, cnt0_out, cnt1_out,
                   edge0, edge1, src0, src1, dst0, dst1, val0, val1, ones_v,
                   x_sp, acc_s, acc_c,
                   sem_in0, sem_in1, sem_sc0, sem_sc1, sem_g):
        c = lax.axis_index("c")
        s = lax.axis_index("s")
        wid = c * _NS + s

        slots = ((edge0, src0, dst0, val0, sem_in0, sem_sc0),
                 (edge1, src1, dst1, val1, sem_in1, sem_sc1))

        pltpu.sync_copy(ones_hbm, ones_v)

        @pl.when(s < _NZCH)
        def _():
            off = s * zlen
            pltpu.sync_copy(zeros_hbm, val0.at[pl.ds(0, zlen)])
            pltpu.sync_copy(val0.at[pl.ds(0, zlen)], acc_s.at[pl.ds(off, zlen)])
            pltpu.sync_copy(val0.at[pl.ds(0, zlen)], acc_c.at[pl.ds(off, zlen)])
            pltpu.sync_copy(x_hbm.at[pl.ds(off, zlen)], val0.at[pl.ds(0, zlen)])
            pltpu.sync_copy(val0.at[pl.ds(0, zlen)], x_sp.at[pl.ds(off, zlen)])

        plsc.subcore_barrier()

        def issue_in(g, slot):
            edgeb, srcb, _, _, semi, _ = slot
            pltpu.async_copy(edge_hbm.at[0, pl.ds(g * _CHUNK, _CHUNK)], srcb, semi)
            pltpu.async_copy(edge_hbm.at[:, pl.ds(g * _CHUNK, _CHUNK)], edgeb, semi)

        def wait_in(g, slot):
            edgeb, srcb, _, _, semi, _ = slot
            pltpu.make_async_copy(edge_hbm.at[0, pl.ds(g * _CHUNK, _CHUNK)], srcb, semi).wait()
            pltpu.make_async_copy(edge_hbm.at[:, pl.ds(g * _CHUNK, _CHUNK)], edgeb, semi).wait()

        def wait_sc(slot):
            _, _, dstb, valb, _, sems = slot
            pltpu.make_async_copy(valb, acc_s.at[dstb], sems).wait()
            pltpu.make_async_copy(ones_v, acc_c.at[dstb], sems).wait()

        def body(k, cur, nxt):
            g = k * _NW + wid
            edgeb, srcb, dstb, valb, semi, sems = cur

            @pl.when(g < n_chunks)
            def _():
                wait_in(g, cur)

                @plsc.parallel_loop(0, _CHUNK, 16, unroll=8)
                def _(j):
                    dstb[pl.ds(j, 16)] = edgeb[1, pl.ds(j, 16)]

                pltpu.async_copy(ones_v, acc_c.at[dstb], sems, add=True)
                pltpu.async_copy(x_sp.at[srcb], valb, sem_g).wait()
                pltpu.async_copy(valb, acc_s.at[dstb], sems, add=True)

            @pl.when(jnp.logical_and(k >= 1, (g - _NW) < n_chunks))
            def _():
                wait_sc(nxt)

            @pl.when((g + _NW) < n_chunks)
            def _():
                issue_in(g + _NW, nxt)

        issue_in(wid, slots[0])

        def pair(p, carry):
            body(2 * p, slots[0], slots[1])
            body(2 * p + 1, slots[1], slots[0])
            return carry
        assert k_iters % 2 == 0
        lax.fori_loop(0, k_iters // 2, pair, 0)

        last = k_iters - 1

        @pl.when((last * _NW + wid) < n_chunks)
        def _():
            wait_sc(slots[last % 2])

        plsc.subcore_barrier()

        @pl.when(s < _NZCH)
        def _():
            off = s * zlen

            pltpu.sync_copy(acc_s.at[pl.ds(off, zlen)], val0.at[pl.ds(0, zlen)])

            @pl.when(c == 0)
            def _():
                pltpu.sync_copy(val0.at[pl.ds(0, zlen)], sum0_out.at[pl.ds(off, zlen)])

            @pl.when(c == 1)
            def _():
                pltpu.sync_copy(val0.at[pl.ds(0, zlen)], sum1_out.at[pl.ds(off, zlen)])

            pltpu.sync_copy(acc_c.at[pl.ds(off, zlen)], val0.at[pl.ds(0, zlen)])

            @pl.when(c == 0)
            def _():
                pltpu.sync_copy(val0.at[pl.ds(0, zlen)], cnt0_out.at[pl.ds(off, zlen)])

            @pl.when(c == 1)
            def _():
                pltpu.sync_copy(val0.at[pl.ds(0, zlen)], cnt1_out.at[pl.ds(off, zlen)])

    return sc_scatter


@functools.lru_cache(maxsize=None)
def _build_tc_dense(N):
    assert (N * 16) % _LANES == 0 and N % 8 == 0
    R = (N * 16) // _LANES

    def body(s0_ref, s1_ref, c0_ref, c1_ref, x_ref, k_ref, wl_ref, bl_ref,
             wr_ref, out_ref):
        ssum = s0_ref[...] + s1_ref[...]
        cnt = c0_ref[...] + c1_ref[...]
        aggr = ssum / jnp.maximum(cnt, 1.0)
        hi = jax.lax.Precision.HIGHEST
        rep = jax.lax.dot(aggr, k_ref[...], precision=hi)
        xrep = jax.lax.dot(x_ref[...], k_ref[...], precision=hi)
        out_ref[...] = jnp.maximum(
            rep * wl_ref[...] + bl_ref[...] + xrep * wr_ref[...], 0.0)

    br = 1600
    grid = (R + br - 1) // br
    vec_spec = pl.BlockSpec((1, _LANES), lambda i: (0, 0))
    col_spec = pl.BlockSpec((br, 8), lambda i: (i, 0))
    return pl.pallas_call(
        body,
        grid=(grid,),
        in_specs=[col_spec, col_spec, col_spec, col_spec, col_spec,
                  pl.BlockSpec((8, _LANES), lambda i: (0, 0)),
                  vec_spec, vec_spec, vec_spec],
        out_specs=pl.BlockSpec((br, _LANES), lambda i: (i, 0)),
        out_shape=jax.ShapeDtypeStruct((R, _LANES), jnp.float32),
    )


def kernel(x, edge_index, W_l, b_l, W_r):
    N, d_in = x.shape
    E = edge_index.shape[1]
    d_out = W_l.shape[0]
    assert d_in == 1 and d_out == 16

    x_flat = x.reshape(N)
    ones_h = jnp.ones((_CHUNK,), jnp.float32)
    zeros_h = jnp.zeros((N // _NZCH,), jnp.float32)
    s0, s1, c0, c1 = _build_sc_scatter(N, E)(x_flat, edge_index, ones_h, zeros_h)

    R = (N * 16) // _LANES
    s0 = s0.reshape(R, 8)
    s1 = s1.reshape(R, 8)
    c0 = c0.reshape(R, 8)
    c1 = c1.reshape(R, 8)
    x2 = x.reshape(R, 8)
    k_mat = jnp.repeat(jnp.eye(8, dtype=jnp.float32), 16, axis=1)
    wl = jnp.tile(W_l.reshape(-1), 8)[None, :]
    bl = jnp.tile(b_l, 8)[None, :]
    wr = jnp.tile(W_r.reshape(-1), 8)[None, :]
    out_flat = _build_tc_dense(N)(s0, s1, c0, c1, x2, k_mat, wl, bl, wr)
    return out_flat.reshape(N, 16)
